# baseline (speedup 1.0000x reference)
import functools

import jax
import jax.numpy as jnp
from jax import lax
from jax.experimental import pallas as pl
from jax.experimental.pallas import tpu as pltpu
from jax.experimental.pallas import tpu_sc as plsc

N, E, D, DE, H = 10000, 320000, 128, 16, 256
NC, NS = 2, 16
NW = NC * NS
EPW = E // NW
CH = 80
NP = 10240
NPT = NP // NS

MB = 4096
EP = 327680

EA = 163840
EB = E - EA
EBP = 159744
NBA = EA // MB
NBB = EBP // MB

_SC_MESH = plsc.VectorSubcoreMesh(core_axis_name="c", subcore_axis_name="s")
_SC_PARAMS = pltpu.CompilerParams(needs_layout_passes=False)


def _chunk_pipeline(nch, fire, proc):
    fire(0, 0)

    def body(k, carry):
        i = 2 * k
        fire(i + 1, 1)
        proc(i, 0)
        fire(i + 2, 0)
        proc(i + 1, 1)
        return carry

    if nch % 2 == 1:
        lax.fori_loop(0, (nch - 1) // 2, body, 0)
        proc(nch - 1, 0)
    else:
        lax.fori_loop(0, (nch - 2) // 2, body, 0)
        fire(nch - 1, 1)
        proc(nch - 2, 0)
        proc(nch - 1, 1)


def _logits_body(row_hbm, col_hbm, s12_hbm, t_hbm, l_hbm,
                 rbuf, cbuf, s1v, s2v, tv, lv):
    c = lax.axis_index("c")
    s = lax.axis_index("s")
    wid = s * NC + c
    base = wid * EPW

    pltpu.sync_copy(row_hbm.at[pl.ds(base, EPW)], rbuf)
    pltpu.sync_copy(col_hbm.at[pl.ds(base, EPW)], cbuf)
    pltpu.sync_copy(s12_hbm.at[0], s1v)
    pltpu.sync_copy(s12_hbm.at[1], s2v)
    pltpu.sync_copy(t_hbm.at[pl.ds(base, EPW)], tv)

    def lbody(j, carry):
        ri = rbuf[pl.ds(j * 16, 16)]
        ci = cbuf[pl.ds(j * 16, 16)]
        a = plsc.load_gather(s1v, [ri])
        b = plsc.load_gather(s2v, [ci])
        lv[pl.ds(j * 16, 16)] = a + b + tv[pl.ds(j * 16, 16)]
        return carry

    lax.fori_loop(0, EPW // 16, lbody, 0)
    pltpu.sync_copy(lv, l_hbm.at[pl.ds(base, EPW)])


_logits_call = functools.partial(
    pl.kernel,
    out_type=jax.ShapeDtypeStruct((EP,), jnp.float32),
    mesh=_SC_MESH,
    compiler_params=_SC_PARAMS,
    scratch_types=[
        pltpu.VMEM((EPW,), jnp.int32),
        pltpu.VMEM((EPW,), jnp.int32),
        pltpu.VMEM((N,), jnp.float32),
        pltpu.VMEM((N,), jnp.float32),
        pltpu.VMEM((EPW,), jnp.float32),
        pltpu.VMEM((EPW,), jnp.float32),
    ],
)(_logits_body)


def _make_gather(e_real, e_pad):
    epw = e_real // NW
    nch = epw // CH

    def body(x_hbm, row_hbm, col_hbm, g0_hbm, g1_hbm,
             rbuf, cbuf, r0a, r1a, r0b, r1b, s0a, s1a, s0b, s1b):
        c = lax.axis_index("c")
        s = lax.axis_index("s")
        wid = s * NC + c
        lbase = wid * epw

        pltpu.sync_copy(row_hbm.at[pl.ds(lbase, epw)], rbuf)
        pltpu.sync_copy(col_hbm.at[pl.ds(lbase, epw)], cbuf)

        rows = ((r0a, r1a, s0a, s1a), (r0b, r1b, s0b, s1b))

        def fire(i, p):
            r0, r1, sg0, sg1 = rows[p]
            pltpu.async_copy(x_hbm.at[rbuf.at[pl.ds(i * CH, CH)]], r0, sg0)
            pltpu.async_copy(x_hbm.at[cbuf.at[pl.ds(i * CH, CH)]], r1, sg1)

        def proc(i, p):
            r0, r1, sg0, sg1 = rows[p]
            pltpu.make_async_copy(
                x_hbm.at[rbuf.at[pl.ds(0, CH)]], r0, sg0).wait()
            pltpu.make_async_copy(
                x_hbm.at[cbuf.at[pl.ds(0, CH)]], r1, sg1).wait()
            e0 = lbase + i * CH
            pltpu.sync_copy(r0, g0_hbm.at[pl.ds(e0, CH)])
            pltpu.sync_copy(r1, g1_hbm.at[pl.ds(e0, CH)])

        _chunk_pipeline(nch, fire, proc)

    return functools.partial(
        pl.kernel,
        out_type=(jax.ShapeDtypeStruct((e_pad, D), jnp.float32),
                  jax.ShapeDtypeStruct((e_pad, D), jnp.float32)),
        mesh=_SC_MESH,
        compiler_params=_SC_PARAMS,
        scratch_types=[
            pltpu.VMEM((epw,), jnp.int32),
            pltpu.VMEM((epw,), jnp.int32),
            pltpu.VMEM((CH, D), jnp.float32),
            pltpu.VMEM((CH, D), jnp.float32),
            pltpu.VMEM((CH, D), jnp.float32),
            pltpu.VMEM((CH, D), jnp.float32),
            pltpu.SemaphoreType.DMA,
            pltpu.SemaphoreType.DMA,
            pltpu.SemaphoreType.DMA,
            pltpu.SemaphoreType.DMA,
        ],
    )(body)


_gather_a = _make_gather(EA, EA)
_gather_b = _make_gather(EB, EBP)


def _make_scatter(e_base, e_real):
    epw = e_real // NW
    nch = epw // CH

    def body(wm_hbm, col_hbm, zero_hbm, part_hbm, acc,
             ia, wa, ib, wb, sia, swa, sib, swb):
        c = lax.axis_index("c")
        s = lax.axis_index("s")
        wid = s * NC + c
        lbase = wid * epw
        gbase = e_base + lbase
        r0 = s * NPT

        pltpu.sync_copy(zero_hbm.at[pl.ds(r0, NPT)], acc.at[pl.ds(r0, NPT)])
        plsc.subcore_barrier()

        bufs = ((ia, wa, sia, swa), (ib, wb, sib, swb))

        def fire(i, p):
            ibuf, wbuf, si, sw = bufs[p]
            pltpu.async_copy(col_hbm.at[pl.ds(gbase + i * CH, CH)], ibuf, si)
            pltpu.async_copy(wm_hbm.at[pl.ds(lbase + i * CH, CH)], wbuf, sw)

        def proc(i, p):
            ibuf, wbuf, si, sw = bufs[p]
            pltpu.make_async_copy(
                col_hbm.at[pl.ds(0, CH)], ibuf, si).wait()
            pltpu.make_async_copy(
                wm_hbm.at[pl.ds(0, CH)], wbuf, sw).wait()
            pltpu.sync_copy(wbuf, acc.at[ibuf], add=True)

        _chunk_pipeline(nch, fire, proc)

        plsc.subcore_barrier()
        pltpu.sync_copy(acc.at[pl.ds(r0, NPT)], part_hbm.at[c, pl.ds(r0, NPT)])

    return functools.partial(
        pl.kernel,
        out_type=jax.ShapeDtypeStruct((NC, NP, D), jnp.float32),
        mesh=_SC_MESH,
        compiler_params=_SC_PARAMS,
        scratch_types=[
            pltpu.VMEM_SHARED((NP, D), jnp.float32),
            pltpu.VMEM((CH,), jnp.int32),
            pltpu.VMEM((CH, D), jnp.float32),
            pltpu.VMEM((CH,), jnp.int32),
            pltpu.VMEM((CH, D), jnp.float32),
            pltpu.SemaphoreType.DMA,
            pltpu.SemaphoreType.DMA,
            pltpu.SemaphoreType.DMA,
            pltpu.SemaphoreType.DMA,
        ],
    )(body)


_scatter_a = _make_scatter(0, EA)
_scatter_b = _make_scatter(EA, EB)


def _pre_kernel(x_ref, eat_ref, va_ref, vb_ref, ve_ref, c0_ref,
                s12_ref, t_ref):
    j = pl.program_id(0)

    @pl.when(j == 0)
    def _():
        s1 = jnp.sum(x_ref[...] * va_ref[...], axis=1, keepdims=True)
        s2 = jnp.sum(x_ref[...] * vb_ref[...], axis=1, keepdims=True)
        s12_ref[...] = jnp.concatenate([s1, s2], axis=1)

    t_ref[...] = (jnp.dot(ve_ref[...], eat_ref[...],
                          preferred_element_type=jnp.float32)
                  + c0_ref[0, 0]).reshape(MB)


def _stats_kernel(l_ref, o_ref):
    v = l_ref[...]
    m = jnp.max(v)
    o_ref[0, 0] = m
    o_ref[0, 1] = jnp.sum(jnp.exp(v - m))


def _gelu(v):
    u = 0.5 * v
    return u + u * lax.erf(v * 0.7071067811865476)


def _mlp_kernel(g0_ref, g1_ref, eat_ref, l_ref, st_ref,
                w1a_ref, w1b_ref, w1e_ref, b1_ref, w2_ref, b2_ref,
                w3_ref, b3_ref, o_ref):
    f32 = jnp.float32
    bf16 = jnp.bfloat16
    ea = jnp.transpose(eat_ref[...])
    h = (jnp.dot(g0_ref[...].astype(bf16), w1a_ref[...],
                 preferred_element_type=f32)
         + jnp.dot(g1_ref[...].astype(bf16), w1b_ref[...],
                   preferred_element_type=f32)
         + jnp.dot(ea.astype(bf16), w1e_ref[...],
                   preferred_element_type=f32)
         + b1_ref[...])
    h = _gelu(h)
    h = _gelu(jnp.dot(h.astype(bf16), w2_ref[...],
                      preferred_element_type=f32) + b2_ref[...])
    msg = (jnp.dot(h.astype(bf16), w3_ref[...], preferred_element_type=f32)
           + b3_ref[...])
    lcol = jnp.transpose(l_ref[...].reshape(1, MB))
    w = jnp.exp(lcol - st_ref[0, 0]) * (1.0 / st_ref[0, 1])
    o_ref[...] = w * msg


def _comb_kernel(pa_ref, pb_ref, o_ref):
    o_ref[...] = (pa_ref[0, :N, :] + pa_ref[1, :N, :]
                  + pb_ref[0, :N, :] + pb_ref[1, :N, :])


def _mlp_call(g0, g1, ea_t, l, stats, weights, nb, blk0):
    w1a, w1b, w1e, b1r, w2b, b2r, w3b, b3r = weights
    return pl.pallas_call(
        _mlp_kernel,
        grid=(nb,),
        in_specs=[
            pl.BlockSpec((MB, D), lambda j: (j, 0)),
            pl.BlockSpec((MB, D), lambda j: (j, 0)),
            pl.BlockSpec((DE, MB), lambda j: (0, j + blk0)),
            pl.BlockSpec((MB,), lambda j: (j + blk0,)),
            pl.BlockSpec((1, 2), lambda j: (0, 0), memory_space=pltpu.SMEM),
            pl.BlockSpec((D, H), lambda j: (0, 0)),
            pl.BlockSpec((D, H), lambda j: (0, 0)),
            pl.BlockSpec((DE, H), lambda j: (0, 0)),
            pl.BlockSpec((1, H), lambda j: (0, 0)),
            pl.BlockSpec((H, H), lambda j: (0, 0)),
            pl.BlockSpec((1, H), lambda j: (0, 0)),
            pl.BlockSpec((H, D), lambda j: (0, 0)),
            pl.BlockSpec((1, D), lambda j: (0, 0)),
        ],
        out_specs=pl.BlockSpec((MB, D), lambda j: (j, 0)),
        out_shape=jax.ShapeDtypeStruct((nb * MB, D), jnp.float32),
    )(g0, g1, ea_t, l, stats, w1a, w1b, w1e, b1r, w2b, b2r, w3b, b3r)


def kernel(x, edge_attr, Wn, bn, We, be, Wa, ba, W1, b1, W2, b2, W3, b3,
           edge_index):
    f32 = jnp.float32
    bf16 = jnp.bfloat16
    row = edge_index[0]
    col = edge_index[1]

    wa1 = Wa[:H, 0]
    wa2 = Wa[H:2 * H, 0]
    wa3 = Wa[2 * H:, 0]
    va = (Wn @ wa1).reshape(1, D)
    vb = (Wn @ wa2).reshape(1, D)
    ve = (We @ wa3).reshape(1, DE)
    c0 = (bn @ wa1 + bn @ wa2 + be @ wa3 + ba[0]).reshape(1, 1)

    weights = (W1[:D].astype(bf16), W1[D:2 * D].astype(bf16),
               W1[2 * D:].astype(bf16), b1.reshape(1, H),
               W2.astype(bf16), b2.reshape(1, H),
               W3.astype(bf16), b3.reshape(1, D))

    ea_t = jnp.pad(edge_attr.T, ((0, 0), (0, EP - E)))

    s12, t = pl.pallas_call(
        _pre_kernel,
        grid=(EP // MB,),
        in_specs=[
            pl.BlockSpec((N, D), lambda j: (0, 0)),
            pl.BlockSpec((DE, MB), lambda j: (0, j)),
            pl.BlockSpec((1, D), lambda j: (0, 0)),
            pl.BlockSpec((1, D), lambda j: (0, 0)),
            pl.BlockSpec((1, DE), lambda j: (0, 0)),
            pl.BlockSpec((1, 1), lambda j: (0, 0), memory_space=pltpu.SMEM),
        ],
        out_specs=[
            pl.BlockSpec((N, 2), lambda j: (0, 0)),
            pl.BlockSpec((MB,), lambda j: (j,)),
        ],
        out_shape=(jax.ShapeDtypeStruct((N, 2), f32),
                   jax.ShapeDtypeStruct((EP,), f32)),
    )(x, ea_t, va, vb, ve, c0)

    g0a, g1a = _gather_a(x, row[:EA], col[:EA])

    l = _logits_call(row, col, s12.T, t)

    stats = pl.pallas_call(
        _stats_kernel,
        in_specs=[pl.BlockSpec((E // 128, 128), lambda: (0, 0))],
        out_specs=pl.BlockSpec((1, 2), lambda: (0, 0),
                               memory_space=pltpu.SMEM),
        out_shape=jax.ShapeDtypeStruct((1, 2), f32),
    )(l[:E].reshape(E // 128, 128))

    g0b, g1b = _gather_b(x, row[EA:], col[EA:])

    wma = _mlp_call(g0a, g1a, ea_t, l, stats, weights, NBA, 0)
    wmb = _mlp_call(g0b, g1b, ea_t, l, stats, weights, NBB, NBA)

    zeros = jnp.zeros((NP, D), f32)
    pa = _scatter_a(wma, col, zeros)
    pb = _scatter_b(wmb, col, zeros)

    out = pl.pallas_call(
        _comb_kernel,
        out_shape=jax.ShapeDtypeStruct((N, D), f32),
    )(pa, pb)

    return out


# speedup vs baseline: 1.0433x; 1.0433x over previous
import functools

import jax
import jax.numpy as jnp
from jax import lax
from jax.experimental import pallas as pl
from jax.experimental.pallas import tpu as pltpu
from jax.experimental.pallas import tpu_sc as plsc

N, E, D, DE, H = 10000, 320000, 128, 16, 256
NC, NS = 2, 16
NW = NC * NS
EPW = E // NW
CH = 80
NP = 10240
NPT = NP // NS

MB = 4096
EP = 327680

E1 = 122880
E2 = 122880
E3 = E - E1 - E2
E3P = 77824
NB1 = E1 // MB
NB3 = E3P // MB

_SC_MESH = plsc.VectorSubcoreMesh(core_axis_name="c", subcore_axis_name="s")
_SC_PARAMS = pltpu.CompilerParams(needs_layout_passes=False)


def _chunk_pipeline(nch, fire, proc):
    fire(0, 0)

    def body(k, carry):
        i = 2 * k
        fire(i + 1, 1)
        proc(i, 0)
        fire(i + 2, 0)
        proc(i + 1, 1)
        return carry

    if nch % 2 == 1:
        lax.fori_loop(0, (nch - 1) // 2, body, 0)
        proc(nch - 1, 0)
    else:
        lax.fori_loop(0, (nch - 2) // 2, body, 0)
        fire(nch - 1, 1)
        proc(nch - 2, 0)
        proc(nch - 1, 1)


def _logits_body(row_hbm, col_hbm, s12_hbm, t_hbm, l_hbm,
                 rbuf, cbuf, s1v, s2v, tv, lv):
    c = lax.axis_index("c")
    s = lax.axis_index("s")
    wid = s * NC + c
    base = wid * EPW

    pltpu.sync_copy(row_hbm.at[pl.ds(base, EPW)], rbuf)
    pltpu.sync_copy(col_hbm.at[pl.ds(base, EPW)], cbuf)
    pltpu.sync_copy(s12_hbm.at[0], s1v)
    pltpu.sync_copy(s12_hbm.at[1], s2v)
    pltpu.sync_copy(t_hbm.at[pl.ds(base, EPW)], tv)

    def lbody(j, carry):
        ri = rbuf[pl.ds(j * 16, 16)]
        ci = cbuf[pl.ds(j * 16, 16)]
        a = plsc.load_gather(s1v, [ri])
        b = plsc.load_gather(s2v, [ci])
        lv[pl.ds(j * 16, 16)] = a + b + tv[pl.ds(j * 16, 16)]
        return carry

    lax.fori_loop(0, EPW // 16, lbody, 0)
    pltpu.sync_copy(lv, l_hbm.at[pl.ds(base, EPW)])


_logits_call = functools.partial(
    pl.kernel,
    out_type=jax.ShapeDtypeStruct((EP,), jnp.float32),
    mesh=_SC_MESH,
    compiler_params=_SC_PARAMS,
    scratch_types=[
        pltpu.VMEM((EPW,), jnp.int32),
        pltpu.VMEM((EPW,), jnp.int32),
        pltpu.VMEM((N,), jnp.float32),
        pltpu.VMEM((N,), jnp.float32),
        pltpu.VMEM((EPW,), jnp.float32),
        pltpu.VMEM((EPW,), jnp.float32),
    ],
)(_logits_body)


def _make_gather(e_real, e_pad):
    epw = e_real // NW
    nch = epw // CH

    def body(x_hbm, row_hbm, col_hbm, g0_hbm, g1_hbm,
             rbuf, cbuf, r0a, r1a, r0b, r1b, s0a, s1a, s0b, s1b):
        c = lax.axis_index("c")
        s = lax.axis_index("s")
        wid = s * NC + c
        lbase = wid * epw

        pltpu.sync_copy(row_hbm.at[pl.ds(lbase, epw)], rbuf)
        pltpu.sync_copy(col_hbm.at[pl.ds(lbase, epw)], cbuf)

        rows = ((r0a, r1a, s0a, s1a), (r0b, r1b, s0b, s1b))

        def fire(i, p):
            r0, r1, sg0, sg1 = rows[p]
            pltpu.async_copy(x_hbm.at[rbuf.at[pl.ds(i * CH, CH)]], r0, sg0)
            pltpu.async_copy(x_hbm.at[cbuf.at[pl.ds(i * CH, CH)]], r1, sg1)

        def proc(i, p):
            r0, r1, sg0, sg1 = rows[p]
            pltpu.make_async_copy(
                x_hbm.at[rbuf.at[pl.ds(0, CH)]], r0, sg0).wait()
            pltpu.make_async_copy(
                x_hbm.at[cbuf.at[pl.ds(0, CH)]], r1, sg1).wait()
            e0 = lbase + i * CH
            pltpu.sync_copy(r0, g0_hbm.at[pl.ds(e0, CH)])
            pltpu.sync_copy(r1, g1_hbm.at[pl.ds(e0, CH)])

        _chunk_pipeline(nch, fire, proc)

    return functools.partial(
        pl.kernel,
        out_type=(jax.ShapeDtypeStruct((e_pad, D), jnp.float32),
                  jax.ShapeDtypeStruct((e_pad, D), jnp.float32)),
        mesh=_SC_MESH,
        compiler_params=_SC_PARAMS,
        scratch_types=[
            pltpu.VMEM((epw,), jnp.int32),
            pltpu.VMEM((epw,), jnp.int32),
            pltpu.VMEM((CH, D), jnp.float32),
            pltpu.VMEM((CH, D), jnp.float32),
            pltpu.VMEM((CH, D), jnp.float32),
            pltpu.VMEM((CH, D), jnp.float32),
            pltpu.SemaphoreType.DMA,
            pltpu.SemaphoreType.DMA,
            pltpu.SemaphoreType.DMA,
            pltpu.SemaphoreType.DMA,
        ],
    )(body)


_gather_ab = _make_gather(E1, E1)
_gather_c = _make_gather(E3, E3P)


def _make_scatter(e_base, e_real):
    epw = e_real // NW
    nch = epw // CH

    def body(wm_hbm, col_hbm, zero_hbm, part_hbm, acc,
             ia, wa, ib, wb, sia, swa, sib, swb):
        c = lax.axis_index("c")
        s = lax.axis_index("s")
        wid = s * NC + c
        lbase = wid * epw
        gbase = e_base + lbase
        r0 = s * NPT

        pltpu.sync_copy(zero_hbm.at[pl.ds(r0, NPT)], acc.at[pl.ds(r0, NPT)])
        plsc.subcore_barrier()

        bufs = ((ia, wa, sia, swa), (ib, wb, sib, swb))

        def fire(i, p):
            ibuf, wbuf, si, sw = bufs[p]
            pltpu.async_copy(col_hbm.at[pl.ds(gbase + i * CH, CH)], ibuf, si)
            pltpu.async_copy(wm_hbm.at[pl.ds(lbase + i * CH, CH)], wbuf, sw)

        def proc(i, p):
            ibuf, wbuf, si, sw = bufs[p]
            pltpu.make_async_copy(
                col_hbm.at[pl.ds(0, CH)], ibuf, si).wait()
            pltpu.make_async_copy(
                wm_hbm.at[pl.ds(0, CH)], wbuf, sw).wait()
            pltpu.sync_copy(wbuf, acc.at[ibuf], add=True)

        _chunk_pipeline(nch, fire, proc)

        plsc.subcore_barrier()
        pltpu.sync_copy(acc.at[pl.ds(r0, NPT)], part_hbm.at[c, pl.ds(r0, NPT)])

    return functools.partial(
        pl.kernel,
        out_type=jax.ShapeDtypeStruct((NC, NP, D), jnp.float32),
        mesh=_SC_MESH,
        compiler_params=_SC_PARAMS,
        scratch_types=[
            pltpu.VMEM_SHARED((NP, D), jnp.float32),
            pltpu.VMEM((CH,), jnp.int32),
            pltpu.VMEM((CH, D), jnp.float32),
            pltpu.VMEM((CH,), jnp.int32),
            pltpu.VMEM((CH, D), jnp.float32),
            pltpu.SemaphoreType.DMA,
            pltpu.SemaphoreType.DMA,
            pltpu.SemaphoreType.DMA,
            pltpu.SemaphoreType.DMA,
        ],
    )(body)


_scatter_a = _make_scatter(0, E1)
_scatter_b = _make_scatter(E1, E2)
_scatter_c = _make_scatter(E1 + E2, E3)


def _pre_kernel(x_ref, eat_ref, va_ref, vb_ref, ve_ref, c0_ref,
                s12_ref, t_ref):
    j = pl.program_id(0)

    @pl.when(j == 0)
    def _():
        s1 = jnp.sum(x_ref[...] * va_ref[...], axis=1, keepdims=True)
        s2 = jnp.sum(x_ref[...] * vb_ref[...], axis=1, keepdims=True)
        s12_ref[...] = jnp.concatenate([s1, s2], axis=1)

    t_ref[...] = (jnp.dot(ve_ref[...], eat_ref[...],
                          preferred_element_type=jnp.float32)
                  + c0_ref[0, 0]).reshape(MB)


def _stats_kernel(l_ref, o_ref):
    v = l_ref[...]
    m = jnp.max(v)
    o_ref[0, 0] = m
    o_ref[0, 1] = jnp.sum(jnp.exp(v - m))


def _gelu(v):
    u = 0.5 * v
    return u + u * lax.erf(v * 0.7071067811865476)


def _mlp_kernel(g0_ref, g1_ref, eat_ref, l_ref, st_ref,
                w1a_ref, w1b_ref, w1e_ref, b1_ref, w2_ref, b2_ref,
                w3_ref, b3_ref, o_ref):
    f32 = jnp.float32
    bf16 = jnp.bfloat16
    ea = jnp.transpose(eat_ref[...])
    h = (jnp.dot(g0_ref[...].astype(bf16), w1a_ref[...],
                 preferred_element_type=f32)
         + jnp.dot(g1_ref[...].astype(bf16), w1b_ref[...],
                   preferred_element_type=f32)
         + jnp.dot(ea.astype(bf16), w1e_ref[...],
                   preferred_element_type=f32)
         + b1_ref[...])
    h = _gelu(h)
    h = _gelu(jnp.dot(h.astype(bf16), w2_ref[...],
                      preferred_element_type=f32) + b2_ref[...])
    msg = (jnp.dot(h.astype(bf16), w3_ref[...], preferred_element_type=f32)
           + b3_ref[...])
    lcol = jnp.transpose(l_ref[...].reshape(1, MB))
    w = jnp.exp(lcol - st_ref[0, 0]) * (1.0 / st_ref[0, 1])
    o_ref[...] = w * msg


def _comb_kernel(pa_ref, pb_ref, pc_ref, o_ref):
    o_ref[...] = (pa_ref[0, :N, :] + pa_ref[1, :N, :]
                  + pb_ref[0, :N, :] + pb_ref[1, :N, :]
                  + pc_ref[0, :N, :] + pc_ref[1, :N, :])


def _mlp_call(g0, g1, ea_t, l, stats, weights, nb, blk0):
    w1a, w1b, w1e, b1r, w2b, b2r, w3b, b3r = weights
    return pl.pallas_call(
        _mlp_kernel,
        grid=(nb,),
        in_specs=[
            pl.BlockSpec((MB, D), lambda j: (j, 0)),
            pl.BlockSpec((MB, D), lambda j: (j, 0)),
            pl.BlockSpec((DE, MB), lambda j: (0, j + blk0)),
            pl.BlockSpec((MB,), lambda j: (j + blk0,)),
            pl.BlockSpec((1, 2), lambda j: (0, 0), memory_space=pltpu.SMEM),
            pl.BlockSpec((D, H), lambda j: (0, 0)),
            pl.BlockSpec((D, H), lambda j: (0, 0)),
            pl.BlockSpec((DE, H), lambda j: (0, 0)),
            pl.BlockSpec((1, H), lambda j: (0, 0)),
            pl.BlockSpec((H, H), lambda j: (0, 0)),
            pl.BlockSpec((1, H), lambda j: (0, 0)),
            pl.BlockSpec((H, D), lambda j: (0, 0)),
            pl.BlockSpec((1, D), lambda j: (0, 0)),
        ],
        out_specs=pl.BlockSpec((MB, D), lambda j: (j, 0)),
        out_shape=jax.ShapeDtypeStruct((nb * MB, D), jnp.float32),
    )(g0, g1, ea_t, l, stats, w1a, w1b, w1e, b1r, w2b, b2r, w3b, b3r)


def kernel(x, edge_attr, Wn, bn, We, be, Wa, ba, W1, b1, W2, b2, W3, b3,
           edge_index):
    f32 = jnp.float32
    bf16 = jnp.bfloat16
    row = edge_index[0]
    col = edge_index[1]

    wa1 = Wa[:H, 0]
    wa2 = Wa[H:2 * H, 0]
    wa3 = Wa[2 * H:, 0]
    va = (Wn @ wa1).reshape(1, D)
    vb = (Wn @ wa2).reshape(1, D)
    ve = (We @ wa3).reshape(1, DE)
    c0 = (bn @ wa1 + bn @ wa2 + be @ wa3 + ba[0]).reshape(1, 1)

    weights = (W1[:D].astype(bf16), W1[D:2 * D].astype(bf16),
               W1[2 * D:].astype(bf16), b1.reshape(1, H),
               W2.astype(bf16), b2.reshape(1, H),
               W3.astype(bf16), b3.reshape(1, D))

    ea_t = jnp.pad(edge_attr.T, ((0, 0), (0, EP - E)))

    s12, t = pl.pallas_call(
        _pre_kernel,
        grid=(EP // MB,),
        in_specs=[
            pl.BlockSpec((N, D), lambda j: (0, 0)),
            pl.BlockSpec((DE, MB), lambda j: (0, j)),
            pl.BlockSpec((1, D), lambda j: (0, 0)),
            pl.BlockSpec((1, D), lambda j: (0, 0)),
            pl.BlockSpec((1, DE), lambda j: (0, 0)),
            pl.BlockSpec((1, 1), lambda j: (0, 0), memory_space=pltpu.SMEM),
        ],
        out_specs=[
            pl.BlockSpec((N, 2), lambda j: (0, 0)),
            pl.BlockSpec((MB,), lambda j: (j,)),
        ],
        out_shape=(jax.ShapeDtypeStruct((N, 2), f32),
                   jax.ShapeDtypeStruct((EP,), f32)),
    )(x, ea_t, va, vb, ve, c0)

    g0a, g1a = _gather_ab(x, row[:E1], col[:E1])

    l = _logits_call(row, col, s12.T, t)

    stats = pl.pallas_call(
        _stats_kernel,
        in_specs=[pl.BlockSpec((E // 128, 128), lambda: (0, 0))],
        out_specs=pl.BlockSpec((1, 2), lambda: (0, 0),
                               memory_space=pltpu.SMEM),
        out_shape=jax.ShapeDtypeStruct((1, 2), f32),
    )(l[:E].reshape(E // 128, 128))

    g0b, g1b = _gather_ab(x, row[E1:E1 + E2], col[E1:E1 + E2])
    wma = _mlp_call(g0a, g1a, ea_t, l, stats, weights, NB1, 0)

    g0c, g1c = _gather_c(x, row[E1 + E2:], col[E1 + E2:])
    wmb = _mlp_call(g0b, g1b, ea_t, l, stats, weights, NB1, NB1)

    zeros = jnp.zeros((NP, D), f32)
    pa = _scatter_a(wma, col, zeros)
    wmc = _mlp_call(g0c, g1c, ea_t, l, stats, weights, NB3, 2 * NB1)

    pb = _scatter_b(wmb, col, zeros)
    pc = _scatter_c(wmc, col, zeros)

    out = pl.pallas_call(
        _comb_kernel,
        out_shape=jax.ShapeDtypeStruct((N, D), f32),
    )(pa, pb, pc)

    return out


# speedup vs baseline: 1.0883x; 1.0431x over previous
import functools

import jax
import jax.numpy as jnp
from jax import lax
from jax.experimental import pallas as pl
from jax.experimental.pallas import tpu as pltpu
from jax.experimental.pallas import tpu_sc as plsc

N, E, D, DE, H = 10000, 320000, 128, 16, 256
NC, NS = 2, 16
NW = NC * NS
EPW = E // NW
CH = 80
NP = 10240
NPT = NP // NS

MB = 4096
MBP = 16384
EP = 327680

E1 = 122880
E2 = 122880
E3 = E - E1 - E2
E3P = 77824
NB1 = E1 // MB
NB3 = E3P // MB

_SC_MESH = plsc.VectorSubcoreMesh(core_axis_name="c", subcore_axis_name="s")
_SC_PARAMS = pltpu.CompilerParams(needs_layout_passes=False)


def _chunk_pipeline(nch, fire, proc):
    fire(0, 0)

    def body(k, carry):
        i = 2 * k
        fire(i + 1, 1)
        proc(i, 0)
        fire(i + 2, 0)
        proc(i + 1, 1)
        return carry

    if nch % 2 == 1:
        lax.fori_loop(0, (nch - 1) // 2, body, 0)
        proc(nch - 1, 0)
    else:
        lax.fori_loop(0, (nch - 2) // 2, body, 0)
        fire(nch - 1, 1)
        proc(nch - 2, 0)
        proc(nch - 1, 1)


def _logits_body(row_hbm, col_hbm, s12_hbm, t_hbm, l_hbm,
                 rbuf, cbuf, s1v, s2v, tv, lv):
    c = lax.axis_index("c")
    s = lax.axis_index("s")
    wid = s * NC + c
    base = wid * EPW

    pltpu.sync_copy(row_hbm.at[pl.ds(base, EPW)], rbuf)
    pltpu.sync_copy(col_hbm.at[pl.ds(base, EPW)], cbuf)
    pltpu.sync_copy(s12_hbm.at[0], s1v)
    pltpu.sync_copy(s12_hbm.at[1], s2v)
    pltpu.sync_copy(t_hbm.at[pl.ds(base, EPW)], tv)

    def lbody(j, carry):
        ri = rbuf[pl.ds(j * 16, 16)]
        ci = cbuf[pl.ds(j * 16, 16)]
        a = plsc.load_gather(s1v, [ri])
        b = plsc.load_gather(s2v, [ci])
        lv[pl.ds(j * 16, 16)] = a + b + tv[pl.ds(j * 16, 16)]
        return carry

    lax.fori_loop(0, EPW // 16, lbody, 0)
    pltpu.sync_copy(lv, l_hbm.at[pl.ds(base, EPW)])


_logits_call = functools.partial(
    pl.kernel,
    out_type=jax.ShapeDtypeStruct((EP,), jnp.float32),
    mesh=_SC_MESH,
    compiler_params=_SC_PARAMS,
    scratch_types=[
        pltpu.VMEM((EPW,), jnp.int32),
        pltpu.VMEM((EPW,), jnp.int32),
        pltpu.VMEM((N,), jnp.float32),
        pltpu.VMEM((N,), jnp.float32),
        pltpu.VMEM((EPW,), jnp.float32),
        pltpu.VMEM((EPW,), jnp.float32),
    ],
)(_logits_body)


def _make_gather(e_real, e_pad):
    epw = e_real // NW
    nch = epw // CH

    def body(x_hbm, row_hbm, col_hbm, g0_hbm, g1_hbm,
             rbuf, cbuf, r0a, r1a, r0b, r1b, s0a, s1a, s0b, s1b):
        c = lax.axis_index("c")
        s = lax.axis_index("s")
        wid = s * NC + c
        lbase = wid * epw

        pltpu.sync_copy(row_hbm.at[pl.ds(lbase, epw)], rbuf)
        pltpu.sync_copy(col_hbm.at[pl.ds(lbase, epw)], cbuf)

        rows = ((r0a, r1a, s0a, s1a), (r0b, r1b, s0b, s1b))

        def fire(i, p):
            r0, r1, sg0, sg1 = rows[p]
            pltpu.async_copy(x_hbm.at[rbuf.at[pl.ds(i * CH, CH)]], r0, sg0)
            pltpu.async_copy(x_hbm.at[cbuf.at[pl.ds(i * CH, CH)]], r1, sg1)

        def proc(i, p):
            r0, r1, sg0, sg1 = rows[p]
            pltpu.make_async_copy(
                x_hbm.at[rbuf.at[pl.ds(0, CH)]], r0, sg0).wait()
            pltpu.make_async_copy(
                x_hbm.at[cbuf.at[pl.ds(0, CH)]], r1, sg1).wait()
            e0 = lbase + i * CH
            pltpu.sync_copy(r0, g0_hbm.at[pl.ds(e0, CH)])
            pltpu.sync_copy(r1, g1_hbm.at[pl.ds(e0, CH)])

        _chunk_pipeline(nch, fire, proc)

    return functools.partial(
        pl.kernel,
        out_type=(jax.ShapeDtypeStruct((e_pad, D), jnp.float32),
                  jax.ShapeDtypeStruct((e_pad, D), jnp.float32)),
        mesh=_SC_MESH,
        compiler_params=_SC_PARAMS,
        scratch_types=[
            pltpu.VMEM((epw,), jnp.int32),
            pltpu.VMEM((epw,), jnp.int32),
            pltpu.VMEM((CH, D), jnp.float32),
            pltpu.VMEM((CH, D), jnp.float32),
            pltpu.VMEM((CH, D), jnp.float32),
            pltpu.VMEM((CH, D), jnp.float32),
            pltpu.SemaphoreType.DMA,
            pltpu.SemaphoreType.DMA,
            pltpu.SemaphoreType.DMA,
            pltpu.SemaphoreType.DMA,
        ],
    )(body)


_gather_ab = _make_gather(E1, E1)
_gather_c = _make_gather(E3, E3P)


def _make_scatter(e_base, e_real):
    epw = e_real // NW
    nch = epw // CH

    def body(wm_hbm, col_hbm, zero_hbm, part_hbm, acc,
             ia, wa, ib, wb, sia, swa, sib, swb):
        c = lax.axis_index("c")
        s = lax.axis_index("s")
        wid = s * NC + c
        lbase = wid * epw
        gbase = e_base + lbase
        r0 = s * NPT

        pltpu.sync_copy(zero_hbm.at[pl.ds(r0, NPT)], acc.at[pl.ds(r0, NPT)])
        plsc.subcore_barrier()

        bufs = ((ia, wa, sia, swa), (ib, wb, sib, swb))

        def fire(i, p):
            ibuf, wbuf, si, sw = bufs[p]
            pltpu.async_copy(col_hbm.at[pl.ds(gbase + i * CH, CH)], ibuf, si)
            pltpu.async_copy(wm_hbm.at[pl.ds(lbase + i * CH, CH)], wbuf, sw)

        def proc(i, p):
            ibuf, wbuf, si, sw = bufs[p]
            pltpu.make_async_copy(
                col_hbm.at[pl.ds(0, CH)], ibuf, si).wait()
            pltpu.make_async_copy(
                wm_hbm.at[pl.ds(0, CH)], wbuf, sw).wait()
            pltpu.sync_copy(wbuf, acc.at[ibuf], add=True)

        _chunk_pipeline(nch, fire, proc)

        plsc.subcore_barrier()
        pltpu.sync_copy(acc.at[pl.ds(r0, NPT)], part_hbm.at[c, pl.ds(r0, NPT)])

    return functools.partial(
        pl.kernel,
        out_type=jax.ShapeDtypeStruct((NC, NP, D), jnp.float32),
        mesh=_SC_MESH,
        compiler_params=_SC_PARAMS,
        scratch_types=[
            pltpu.VMEM_SHARED((NP, D), jnp.float32),
            pltpu.VMEM((CH,), jnp.int32),
            pltpu.VMEM((CH, D), jnp.float32),
            pltpu.VMEM((CH,), jnp.int32),
            pltpu.VMEM((CH, D), jnp.float32),
            pltpu.SemaphoreType.DMA,
            pltpu.SemaphoreType.DMA,
            pltpu.SemaphoreType.DMA,
            pltpu.SemaphoreType.DMA,
        ],
    )(body)


_scatter_a = _make_scatter(0, E1)
_scatter_b = _make_scatter(E1, E2)
_scatter_c = _make_scatter(E1 + E2, E3)


def _pre_kernel(x_ref, eat_ref, va_ref, vb_ref, ve_ref, c0_ref,
                s12_ref, t_ref):
    j = pl.program_id(0)

    @pl.when(j == 0)
    def _():
        s1 = jnp.sum(x_ref[...] * va_ref[...], axis=1, keepdims=True)
        s2 = jnp.sum(x_ref[...] * vb_ref[...], axis=1, keepdims=True)
        s12_ref[...] = jnp.concatenate([s1, s2], axis=1)

    t_ref[...] = (jnp.dot(ve_ref[...], eat_ref[...],
                          preferred_element_type=jnp.float32)
                  + c0_ref[0, 0]).reshape(MBP)


def _stats_kernel(l_ref, o_ref):
    v = l_ref[...]
    m = jnp.max(v)
    o_ref[0, 0] = m
    o_ref[0, 1] = jnp.sum(jnp.exp(v - m))


def _gelu(v):
    u = 0.5 * v
    return u + u * lax.erf(v * 0.7071067811865476)


def _mlp_kernel(g0_ref, g1_ref, eat_ref, l_ref, st_ref,
                w1a_ref, w1b_ref, w1e_ref, b1_ref, w2_ref, b2_ref,
                w3_ref, b3_ref, o_ref):
    f32 = jnp.float32
    bf16 = jnp.bfloat16
    ea = jnp.transpose(eat_ref[...])
    h = (jnp.dot(g0_ref[...].astype(bf16), w1a_ref[...],
                 preferred_element_type=f32)
         + jnp.dot(g1_ref[...].astype(bf16), w1b_ref[...],
                   preferred_element_type=f32)
         + jnp.dot(ea.astype(bf16), w1e_ref[...],
                   preferred_element_type=f32)
         + b1_ref[...])
    h = _gelu(h)
    h = _gelu(jnp.dot(h.astype(bf16), w2_ref[...],
                      preferred_element_type=f32) + b2_ref[...])
    msg = (jnp.dot(h.astype(bf16), w3_ref[...], preferred_element_type=f32)
           + b3_ref[...])
    lcol = jnp.transpose(l_ref[...].reshape(1, MB))
    w = jnp.exp(lcol - st_ref[0, 0]) * (1.0 / st_ref[0, 1])
    o_ref[...] = w * msg


def _comb_kernel(pa_ref, pb_ref, pc_ref, o_ref):
    o_ref[...] = (pa_ref[0, :N, :] + pa_ref[1, :N, :]
                  + pb_ref[0, :N, :] + pb_ref[1, :N, :]
                  + pc_ref[0, :N, :] + pc_ref[1, :N, :])


def _mlp_call(g0, g1, ea_t, l, stats, weights, nb, blk0):
    w1a, w1b, w1e, b1r, w2b, b2r, w3b, b3r = weights
    return pl.pallas_call(
        _mlp_kernel,
        grid=(nb,),
        in_specs=[
            pl.BlockSpec((MB, D), lambda j: (j, 0)),
            pl.BlockSpec((MB, D), lambda j: (j, 0)),
            pl.BlockSpec((DE, MB), lambda j: (0, j + blk0)),
            pl.BlockSpec((MB,), lambda j: (j + blk0,)),
            pl.BlockSpec((1, 2), lambda j: (0, 0), memory_space=pltpu.SMEM),
            pl.BlockSpec((D, H), lambda j: (0, 0)),
            pl.BlockSpec((D, H), lambda j: (0, 0)),
            pl.BlockSpec((DE, H), lambda j: (0, 0)),
            pl.BlockSpec((1, H), lambda j: (0, 0)),
            pl.BlockSpec((H, H), lambda j: (0, 0)),
            pl.BlockSpec((1, H), lambda j: (0, 0)),
            pl.BlockSpec((H, D), lambda j: (0, 0)),
            pl.BlockSpec((1, D), lambda j: (0, 0)),
        ],
        out_specs=pl.BlockSpec((MB, D), lambda j: (j, 0)),
        out_shape=jax.ShapeDtypeStruct((nb * MB, D), jnp.float32),
    )(g0, g1, ea_t, l, stats, w1a, w1b, w1e, b1r, w2b, b2r, w3b, b3r)


def kernel(x, edge_attr, Wn, bn, We, be, Wa, ba, W1, b1, W2, b2, W3, b3,
           edge_index):
    f32 = jnp.float32
    bf16 = jnp.bfloat16
    row = edge_index[0]
    col = edge_index[1]

    wa1 = Wa[:H, 0]
    wa2 = Wa[H:2 * H, 0]
    wa3 = Wa[2 * H:, 0]
    va = (Wn @ wa1).reshape(1, D)
    vb = (Wn @ wa2).reshape(1, D)
    ve = (We @ wa3).reshape(1, DE)
    c0 = (bn @ wa1 + bn @ wa2 + be @ wa3 + ba[0]).reshape(1, 1)

    weights = (W1[:D].astype(bf16), W1[D:2 * D].astype(bf16),
               W1[2 * D:].astype(bf16), b1.reshape(1, H),
               W2.astype(bf16), b2.reshape(1, H),
               W3.astype(bf16), b3.reshape(1, D))

    ea_t = jnp.pad(edge_attr.T, ((0, 0), (0, EP - E)))

    s12, t = pl.pallas_call(
        _pre_kernel,
        grid=(EP // MBP,),
        in_specs=[
            pl.BlockSpec((N, D), lambda j: (0, 0)),
            pl.BlockSpec((DE, MBP), lambda j: (0, j)),
            pl.BlockSpec((1, D), lambda j: (0, 0)),
            pl.BlockSpec((1, D), lambda j: (0, 0)),
            pl.BlockSpec((1, DE), lambda j: (0, 0)),
            pl.BlockSpec((1, 1), lambda j: (0, 0), memory_space=pltpu.SMEM),
        ],
        out_specs=[
            pl.BlockSpec((N, 2), lambda j: (0, 0)),
            pl.BlockSpec((MBP,), lambda j: (j,)),
        ],
        out_shape=(jax.ShapeDtypeStruct((N, 2), f32),
                   jax.ShapeDtypeStruct((EP,), f32)),
    )(x, ea_t, va, vb, ve, c0)

    g0a, g1a = _gather_ab(x, row[:E1], col[:E1])

    l = _logits_call(row, col, s12.T, t)

    stats = pl.pallas_call(
        _stats_kernel,
        in_specs=[pl.BlockSpec((E // 128, 128), lambda: (0, 0))],
        out_specs=pl.BlockSpec((1, 2), lambda: (0, 0),
                               memory_space=pltpu.SMEM),
        out_shape=jax.ShapeDtypeStruct((1, 2), f32),
    )(l[:E].reshape(E // 128, 128))

    g0b, g1b = _gather_ab(x, row[E1:E1 + E2], col[E1:E1 + E2])
    wma = _mlp_call(g0a, g1a, ea_t, l, stats, weights, NB1, 0)

    g0c, g1c = _gather_c(x, row[E1 + E2:], col[E1 + E2:])
    wmb = _mlp_call(g0b, g1b, ea_t, l, stats, weights, NB1, NB1)

    zeros = jnp.zeros((NP, D), f32)
    pa = _scatter_a(wma, col, zeros)
    wmc = _mlp_call(g0c, g1c, ea_t, l, stats, weights, NB3, 2 * NB1)

    pb = _scatter_b(wmb, col, zeros)
    pc = _scatter_c(wmc, col, zeros)

    out = pl.pallas_call(
        _comb_kernel,
        out_shape=jax.ShapeDtypeStruct((N, D), f32),
    )(pa, pb, pc)

    return out
